# baseline (speedup 1.0000x reference)
import jax
import jax.numpy as jnp
from jax import lax
from jax.experimental import pallas as pl
from jax.experimental.pallas import tpu as pltpu
from jax.experimental.pallas import tpu_sc as plsc

IMG = 608.0
NCLS = 80
NFM = 3
A = 3
FM = (19.0, 38.0, 76.0)
LAST = (0, 1083, 5415)
SCALED_ANCH = (
    ((3.625, 2.8125), (4.875, 6.1875), (11.65625, 10.1875)),
    ((1.875, 3.8125), (3.875, 2.8125), (3.6875, 7.4375)),
    ((1.25, 1.625), (2.0, 2.875), (4.125, 2.875)),
)
NT = 50
NPAD = 64
BS = 16
NANCH = 22743
NALL = BS * NANCH
APAD = 23552
BA = 2048
NHIT = A * NFM * NPAD


def _assign_vec(t1, t2, t3, t4, use_floor):
    best = None
    best_pos = None
    bases = []
    ious = []
    for m in range(NFM):
        fm = FM[m]
        gw = t3 * fm
        gh = t4 * fm
        if use_floor:
            gi = jnp.floor(t1 * fm).astype(jnp.int32)
            gj = jnp.floor(t2 * fm).astype(jnp.int32)
        else:
            gi = (t1 * fm).astype(jnp.int32)
            gj = (t2 * fm).astype(jnp.int32)
        base = LAST[m] + A * (gi * gj)
        bases.append(base)
        row = []
        for i in range(A):
            aw, ah = SCALED_ANCH[m][i]
            inter = jnp.minimum(gw, aw) * jnp.minimum(gh, ah)
            union = gw * gh + (aw * ah) - inter
            iou = inter / (union + 1e-16)
            row.append(iou)
            cand_pos = base + i
            if best is None:
                best, best_pos = iou, cand_pos
            else:
                upd = iou > best
                best = jnp.where(upd, iou, best)
                best_pos = jnp.where(upd, cand_pos, best_pos)
        ious.append(row)
    return best_pos, bases, best, ious


def _extract_kernel(targets_ref, pred_ref, conf_ref, cand_ref, s1p_ref):
    g = pl.program_id(0)
    f32 = jnp.float32
    data = pred_ref[...]
    a_iota = lax.broadcasted_iota(jnp.int32, (BS, BA), 1) + g * BA
    valid_a = a_iota < NANCH
    lane3 = lax.broadcasted_iota(jnp.int32, data.shape, 2)
    conf = jnp.sum(jnp.where(lane3 == 4, data, 0.0), axis=2)
    conf_m = jnp.where(valid_a, conf, 0.0)
    conf_ref[...] = conf_m.reshape(BS, BA // 128, 128)

    clipv = lambda p: jnp.clip(p, 1e-7, 1.0 - 1e-7)
    terms = jnp.where(valid_a, -jnp.log(1.0 - clipv(conf)), 0.0)
    t3 = terms.reshape(BS, BA // 128, 128)
    psum = jnp.sum(t3, axis=(0, 1))
    p8 = jnp.broadcast_to(psum[None, :], (8, 128)) * f32(0.125)

    targets_p = [targets_ref[c] for c in range(5)]
    pos, _, _, _ = _assign_vec(targets_p[1], targets_p[2], targets_p[3],
                               targets_p[4], True)

    a3 = lax.broadcasted_iota(jnp.int32, data.shape, 1) + g * BA
    dmask = jnp.where(a3 < NANCH, data, 0.0)

    @pl.when(g == 0)
    def _init():
        cand_ref[...] = jnp.zeros((BS, NPAD, 85), f32)
        s1p_ref[...] = jnp.zeros((8, 128), f32)

    s1p_ref[...] += p8
    for b in range(BS):
        a_row = lax.broadcasted_iota(jnp.int32, (NPAD, BA), 1) + g * BA
        onehot = (a_row == pos[b][:, None]).astype(f32)
        cand_ref[b] += lax.dot_general(
            onehot, dmask[b],
            (((1,), (0,)), ((), ())),
            preferred_element_type=f32)


def _extract(targets_t, prediction):
    grid = -(-NANCH // BA)
    return pl.pallas_call(
        _extract_kernel,
        grid=(grid,),
        in_specs=[
            pl.BlockSpec((5, BS, NPAD), lambda g: (0, 0, 0)),
            pl.BlockSpec((BS, BA, 85), lambda g: (0, g, 0)),
        ],
        out_specs=[
            pl.BlockSpec((BS, BA // 128, 128), lambda g: (0, g, 0)),
            pl.BlockSpec((BS, NPAD, 85), lambda g: (0, 0, 0)),
            pl.BlockSpec((8, 128), lambda g: (0, 0)),
        ],
        out_shape=[
            jax.ShapeDtypeStruct((BS, APAD // 128, 128), jnp.float32),
            jax.ShapeDtypeStruct((BS, NPAD, 85), jnp.float32),
            jax.ShapeDtypeStruct((8, 128), jnp.float32),
        ],
    )(targets_t, prediction)


def _sc_gather_kernel(conf_hbm, targets_hbm, hitconf_out,
                      tgt_v, hitw_idx_v, hitw_v, sem):
    wid = lax.axis_index("s") * 2 + lax.axis_index("c")

    @pl.when(wid < BS)
    def _sparse():
        b = wid
        pltpu.sync_copy(targets_hbm.at[:, b, :], tgt_v)
        word0 = b * APAD
        pad = jnp.full((16,), 0, jnp.int32)
        for kpad in range(NHIT, 640, 16):
            hitw_idx_v[pl.ds(kpad, 16)] = pad
        for chunk in range(NPAD // 16):
            sl = pl.ds(chunk * 16, 16)
            t1 = tgt_v[1, sl]
            t2 = tgt_v[2, sl]
            t3 = tgt_v[3, sl]
            t4 = tgt_v[4, sl]
            _, bases, _, _ = _assign_vec(t1, t2, t3, t4, False)
            for m in range(NFM):
                for i in range(A):
                    k = (m * A + i) * NPAD + chunk * 16
                    hitw_idx_v[pl.ds(k, 16)] = (
                        word0 + jnp.clip(bases[m] + i, 0, NANCH - 1))
        pltpu.async_copy(conf_hbm.at[hitw_idx_v], hitw_v, sem).wait()
        pltpu.sync_copy(hitw_v, hitconf_out.at[b])


def _sc_gather(conf_flat, targets_t):
    mesh = plsc.VectorSubcoreMesh(core_axis_name="c", subcore_axis_name="s")
    kfn = pl.kernel(
        _sc_gather_kernel,
        mesh=mesh,
        compiler_params=pltpu.CompilerParams(use_tc_tiling_on_sc=False),
        out_type=[
            jax.ShapeDtypeStruct((BS, 640), jnp.float32),
        ],
        scratch_types=[
            pltpu.VMEM((5, NPAD), jnp.float32),
            pltpu.VMEM((640,), jnp.int32),
            pltpu.VMEM((640,), jnp.float32),
            pltpu.SemaphoreType.DMA,
        ],
    )
    return kfn(conf_flat, targets_t)


def _reduce_kernel(targets_ref, cand_ref, hitconf_ref, s1p_ref, out_ref):
    f32 = jnp.float32
    targets_p = [targets_ref[c] for c in range(5)]
    valid = (targets_p[0] + targets_p[1] + targets_p[2] +
             targets_p[3] + targets_p[4]) != 0.0
    pos, bases, tconf, ious = _assign_vec(
        targets_p[1], targets_p[2], targets_p[3], targets_p[4], True)
    cls = targets_p[0].astype(jnp.int32)

    clipv = lambda p: jnp.clip(p, 1e-7, 1.0 - 1e-7)
    C0 = -jnp.log(f32(1.0) - clipv(f32(0.0)))
    Nf = f32(NALL)

    n_iota2 = lax.broadcasted_iota(jnp.int32, (BS, NPAD, NPAD), 2)
    n_iota1 = lax.broadcasted_iota(jnp.int32, (BS, NPAD, NPAD), 1)
    eqpos = pos[:, :, None] == pos[:, None, :]
    overw = eqpos & (n_iota2 > n_iota1) & valid[:, None, :]
    is_last = valid & jnp.logical_not(jnp.any(overw, axis=2))
    eqcls = cls[:, :, None] == cls[:, None, :]
    earlier_same = eqpos & eqcls & (n_iota2 < n_iota1) & valid[:, None, :]
    cls_first = valid & jnp.logical_not(jnp.any(earlier_same, axis=2))
    wl = is_last.astype(f32)
    wcf = cls_first.astype(f32)

    cand = cand_ref[...]
    lane = lax.broadcasted_iota(jnp.int32, cand.shape, 2)
    pick = lambda c: jnp.sum(jnp.where(lane == c, cand, 0.0), axis=2)
    xg, yg, wg, hg, cg = pick(0), pick(1), pick(2), pick(3), pick(4)

    wwh = 2.0 - targets_p[3] * targets_p[4]
    tx = targets_p[1] * IMG
    ty = targets_p[2] * IMG
    tw = targets_p[3] * IMG
    th = targets_p[4] * IMG
    sxywh = jnp.sum(wl * ((xg * wwh - tx * wwh) ** 2 +
                          (yg * wwh - ty * wwh) ** 2 +
                          (wg * wwh - tw * wwh) ** 2 +
                          (hg * wwh - th * wwh) ** 2))

    cgc = clipv(cg)
    mask_corr = jnp.sum(wl * (-(tconf * jnp.log(cgc) +
                                (1.0 - tconf) * jnp.log(1.0 - cgc)) - C0))

    pc = clipv(cand)
    l1m = jnp.log(1.0 - pc)
    rowsum = jnp.sum(jnp.where(lane >= 5, -l1m, 0.0), axis=2)
    selmask = lane == (5 + cls)[:, :, None]
    selterm = jnp.sum(jnp.where(selmask, l1m - jnp.log(pc), 0.0), axis=2)
    clsnum = jnp.sum(wl * rowsum) + jnp.sum(wcf * selterm)
    nmask = jnp.sum(wl)

    S1 = jnp.sum(s1p_ref[...])

    noobj_corr = f32(0.0)
    for m in range(NFM):
        hitk = jnp.concatenate(
            [((ious[m][i] > 0.5) & valid).astype(jnp.int32)
             for i in range(A)], axis=1) > 0
        idxk = jnp.concatenate(
            [bases[m] + i for i in range(A)], axis=1)
        K = A * NPAD
        k2 = lax.broadcasted_iota(jnp.int32, (BS, K, K), 2)
        k1 = lax.broadcasted_iota(jnp.int32, (BS, K, K), 1)
        eqi = idxk[:, :, None] == idxk[:, None, :]
        earlier_hit = eqi & (k2 < k1) & hitk[:, None, :]
        hit_keep = hitk & jnp.logical_not(jnp.any(earlier_hit, axis=2))
        hconf = jnp.concatenate(
            [hitconf_ref[:, (m * A + i) * NPAD:(m * A + i + 1) * NPAD]
             for i in range(A)], axis=1)
        noobj_corr = noobj_corr + jnp.sum(
            hit_keep.astype(f32) * (C0 + jnp.log(1.0 - clipv(hconf))))

    loss = (sxywh / Nf
            + (Nf * C0 + mask_corr) / Nf
            + 0.5 * (S1 + noobj_corr) / Nf
            + clsnum / (nmask * NCLS))
    out_ref[...] = jnp.full((8, 128), loss, jnp.float32)


def _reduce(targets_t, cand, hitconf, s1p):
    return pl.pallas_call(
        _reduce_kernel,
        out_shape=jax.ShapeDtypeStruct((8, 128), jnp.float32),
    )(targets_t, cand, hitconf, s1p)


def kernel(prediction, targets, stride):
    targets_p = jnp.pad(targets, ((0, 0), (0, NPAD - NT), (0, 0)))
    targets_t = jnp.transpose(targets_p, (2, 0, 1))
    conf3, cand, s1p = _extract(targets_t, prediction)
    conf_flat = conf3.reshape(BS * APAD)
    hitconf, = _sc_gather(conf_flat, targets_t)
    out = _reduce(targets_t, cand, hitconf, s1p)
    return out[0, 0]


# speedup vs baseline: 1.0044x; 1.0044x over previous
import jax
import jax.numpy as jnp
from jax import lax
from jax.experimental import pallas as pl
from jax.experimental.pallas import tpu as pltpu
from jax.experimental.pallas import tpu_sc as plsc

IMG = 608.0
NCLS = 80
NFM = 3
A = 3
FM = (19.0, 38.0, 76.0)
LAST = (0, 1083, 5415)
SCALED_ANCH = (
    ((3.625, 2.8125), (4.875, 6.1875), (11.65625, 10.1875)),
    ((1.875, 3.8125), (3.875, 2.8125), (3.6875, 7.4375)),
    ((1.25, 1.625), (2.0, 2.875), (4.125, 2.875)),
)
NT = 50
NPAD = 64
BS = 16
NANCH = 22743
NALL = BS * NANCH
APAD = 23552
BA = 1024
NHIT = A * NFM * NPAD


def _assign_vec(t1, t2, t3, t4, use_floor):
    best = None
    best_pos = None
    bases = []
    ious = []
    for m in range(NFM):
        fm = FM[m]
        gw = t3 * fm
        gh = t4 * fm
        if use_floor:
            gi = jnp.floor(t1 * fm).astype(jnp.int32)
            gj = jnp.floor(t2 * fm).astype(jnp.int32)
        else:
            gi = (t1 * fm).astype(jnp.int32)
            gj = (t2 * fm).astype(jnp.int32)
        base = LAST[m] + A * (gi * gj)
        bases.append(base)
        row = []
        for i in range(A):
            aw, ah = SCALED_ANCH[m][i]
            inter = jnp.minimum(gw, aw) * jnp.minimum(gh, ah)
            union = gw * gh + (aw * ah) - inter
            iou = inter / (union + 1e-16)
            row.append(iou)
            cand_pos = base + i
            if best is None:
                best, best_pos = iou, cand_pos
            else:
                upd = iou > best
                best = jnp.where(upd, iou, best)
                best_pos = jnp.where(upd, cand_pos, best_pos)
        ious.append(row)
    return best_pos, bases, best, ious


def _extract_kernel(targets_ref, pred_ref, conf_ref, cand_ref, s1p_ref):
    g = pl.program_id(0)
    f32 = jnp.float32
    data = pred_ref[...]
    a_iota = lax.broadcasted_iota(jnp.int32, (BS, BA), 1) + g * BA
    valid_a = a_iota < NANCH
    lane3 = lax.broadcasted_iota(jnp.int32, data.shape, 2)
    conf = jnp.sum(jnp.where(lane3 == 4, data, 0.0), axis=2)
    conf_m = jnp.where(valid_a, conf, 0.0)
    conf_ref[...] = conf_m.reshape(BS, BA // 128, 128)

    clipv = lambda p: jnp.clip(p, 1e-7, 1.0 - 1e-7)
    terms = jnp.where(valid_a, -jnp.log(1.0 - clipv(conf)), 0.0)
    t3 = terms.reshape(BS, BA // 128, 128)
    psum = jnp.sum(t3, axis=(0, 1))
    p8 = jnp.broadcast_to(psum[None, :], (8, 128)) * f32(0.125)

    targets_p = [targets_ref[c] for c in range(5)]
    pos, _, _, _ = _assign_vec(targets_p[1], targets_p[2], targets_p[3],
                               targets_p[4], True)

    a3 = lax.broadcasted_iota(jnp.int32, data.shape, 1) + g * BA
    dmask = jnp.where(a3 < NANCH, data, 0.0)

    @pl.when(g == 0)
    def _init():
        cand_ref[...] = jnp.zeros((BS, NPAD, 85), f32)
        s1p_ref[...] = jnp.zeros((8, 128), f32)

    s1p_ref[...] += p8
    for b in range(BS):
        a_row = lax.broadcasted_iota(jnp.int32, (NPAD, BA), 1) + g * BA
        onehot = (a_row == pos[b][:, None]).astype(f32)
        cand_ref[b] += lax.dot_general(
            onehot, dmask[b],
            (((1,), (0,)), ((), ())),
            preferred_element_type=f32)


def _extract(targets_t, prediction):
    grid = -(-NANCH // BA)
    return pl.pallas_call(
        _extract_kernel,
        grid=(grid,),
        in_specs=[
            pl.BlockSpec((5, BS, NPAD), lambda g: (0, 0, 0)),
            pl.BlockSpec((BS, BA, 85), lambda g: (0, g, 0)),
        ],
        out_specs=[
            pl.BlockSpec((BS, BA // 128, 128), lambda g: (0, g, 0)),
            pl.BlockSpec((BS, NPAD, 85), lambda g: (0, 0, 0)),
            pl.BlockSpec((8, 128), lambda g: (0, 0)),
        ],
        out_shape=[
            jax.ShapeDtypeStruct((BS, APAD // 128, 128), jnp.float32),
            jax.ShapeDtypeStruct((BS, NPAD, 85), jnp.float32),
            jax.ShapeDtypeStruct((8, 128), jnp.float32),
        ],
    )(targets_t, prediction)


def _sc_gather_kernel(conf_hbm, targets_hbm, hitconf_out,
                      tgt_v, hitw_idx_v, hitw_v, sem):
    wid = lax.axis_index("s") * 2 + lax.axis_index("c")

    @pl.when(wid < BS)
    def _sparse():
        b = wid
        pltpu.sync_copy(targets_hbm.at[:, b, :], tgt_v)
        word0 = b * APAD
        pad = jnp.full((16,), 0, jnp.int32)
        for kpad in range(NHIT, 640, 16):
            hitw_idx_v[pl.ds(kpad, 16)] = pad
        for chunk in range(NPAD // 16):
            sl = pl.ds(chunk * 16, 16)
            t1 = tgt_v[1, sl]
            t2 = tgt_v[2, sl]
            t3 = tgt_v[3, sl]
            t4 = tgt_v[4, sl]
            _, bases, _, _ = _assign_vec(t1, t2, t3, t4, False)
            for m in range(NFM):
                for i in range(A):
                    k = (m * A + i) * NPAD + chunk * 16
                    hitw_idx_v[pl.ds(k, 16)] = (
                        word0 + jnp.clip(bases[m] + i, 0, NANCH - 1))
        pltpu.async_copy(conf_hbm.at[hitw_idx_v], hitw_v, sem).wait()
        pltpu.sync_copy(hitw_v, hitconf_out.at[b])


def _sc_gather(conf_flat, targets_t):
    mesh = plsc.VectorSubcoreMesh(core_axis_name="c", subcore_axis_name="s")
    kfn = pl.kernel(
        _sc_gather_kernel,
        mesh=mesh,
        compiler_params=pltpu.CompilerParams(use_tc_tiling_on_sc=False),
        out_type=[
            jax.ShapeDtypeStruct((BS, 640), jnp.float32),
        ],
        scratch_types=[
            pltpu.VMEM((5, NPAD), jnp.float32),
            pltpu.VMEM((640,), jnp.int32),
            pltpu.VMEM((640,), jnp.float32),
            pltpu.SemaphoreType.DMA,
        ],
    )
    return kfn(conf_flat, targets_t)


def _reduce_kernel(targets_ref, cand_ref, hitconf_ref, s1p_ref, out_ref):
    f32 = jnp.float32
    targets_p = [targets_ref[c] for c in range(5)]
    valid = (targets_p[0] + targets_p[1] + targets_p[2] +
             targets_p[3] + targets_p[4]) != 0.0
    pos, bases, tconf, ious = _assign_vec(
        targets_p[1], targets_p[2], targets_p[3], targets_p[4], True)
    cls = targets_p[0].astype(jnp.int32)

    clipv = lambda p: jnp.clip(p, 1e-7, 1.0 - 1e-7)
    C0 = -jnp.log(f32(1.0) - clipv(f32(0.0)))
    Nf = f32(NALL)

    n_iota2 = lax.broadcasted_iota(jnp.int32, (BS, NPAD, NPAD), 2)
    n_iota1 = lax.broadcasted_iota(jnp.int32, (BS, NPAD, NPAD), 1)
    eqpos = pos[:, :, None] == pos[:, None, :]
    overw = eqpos & (n_iota2 > n_iota1) & valid[:, None, :]
    is_last = valid & jnp.logical_not(jnp.any(overw, axis=2))
    eqcls = cls[:, :, None] == cls[:, None, :]
    earlier_same = eqpos & eqcls & (n_iota2 < n_iota1) & valid[:, None, :]
    cls_first = valid & jnp.logical_not(jnp.any(earlier_same, axis=2))
    wl = is_last.astype(f32)
    wcf = cls_first.astype(f32)

    cand = cand_ref[...]
    lane = lax.broadcasted_iota(jnp.int32, cand.shape, 2)
    pick = lambda c: jnp.sum(jnp.where(lane == c, cand, 0.0), axis=2)
    xg, yg, wg, hg, cg = pick(0), pick(1), pick(2), pick(3), pick(4)

    wwh = 2.0 - targets_p[3] * targets_p[4]
    tx = targets_p[1] * IMG
    ty = targets_p[2] * IMG
    tw = targets_p[3] * IMG
    th = targets_p[4] * IMG
    sxywh = jnp.sum(wl * ((xg * wwh - tx * wwh) ** 2 +
                          (yg * wwh - ty * wwh) ** 2 +
                          (wg * wwh - tw * wwh) ** 2 +
                          (hg * wwh - th * wwh) ** 2))

    cgc = clipv(cg)
    mask_corr = jnp.sum(wl * (-(tconf * jnp.log(cgc) +
                                (1.0 - tconf) * jnp.log(1.0 - cgc)) - C0))

    pc = clipv(cand)
    l1m = jnp.log(1.0 - pc)
    rowsum = jnp.sum(jnp.where(lane >= 5, -l1m, 0.0), axis=2)
    selmask = lane == (5 + cls)[:, :, None]
    selterm = jnp.sum(jnp.where(selmask, l1m - jnp.log(pc), 0.0), axis=2)
    clsnum = jnp.sum(wl * rowsum) + jnp.sum(wcf * selterm)
    nmask = jnp.sum(wl)

    S1 = jnp.sum(s1p_ref[...])

    noobj_corr = f32(0.0)
    for m in range(NFM):
        hitk = jnp.concatenate(
            [((ious[m][i] > 0.5) & valid).astype(jnp.int32)
             for i in range(A)], axis=1) > 0
        idxk = jnp.concatenate(
            [bases[m] + i for i in range(A)], axis=1)
        K = A * NPAD
        k2 = lax.broadcasted_iota(jnp.int32, (BS, K, K), 2)
        k1 = lax.broadcasted_iota(jnp.int32, (BS, K, K), 1)
        eqi = idxk[:, :, None] == idxk[:, None, :]
        earlier_hit = eqi & (k2 < k1) & hitk[:, None, :]
        hit_keep = hitk & jnp.logical_not(jnp.any(earlier_hit, axis=2))
        hconf = jnp.concatenate(
            [hitconf_ref[:, (m * A + i) * NPAD:(m * A + i + 1) * NPAD]
             for i in range(A)], axis=1)
        noobj_corr = noobj_corr + jnp.sum(
            hit_keep.astype(f32) * (C0 + jnp.log(1.0 - clipv(hconf))))

    loss = (sxywh / Nf
            + (Nf * C0 + mask_corr) / Nf
            + 0.5 * (S1 + noobj_corr) / Nf
            + clsnum / (nmask * NCLS))
    out_ref[...] = jnp.full((8, 128), loss, jnp.float32)


def _reduce(targets_t, cand, hitconf, s1p):
    return pl.pallas_call(
        _reduce_kernel,
        out_shape=jax.ShapeDtypeStruct((8, 128), jnp.float32),
    )(targets_t, cand, hitconf, s1p)


def kernel(prediction, targets, stride):
    targets_p = jnp.pad(targets, ((0, 0), (0, NPAD - NT), (0, 0)))
    targets_t = jnp.transpose(targets_p, (2, 0, 1))
    conf3, cand, s1p = _extract(targets_t, prediction)
    conf_flat = conf3.reshape(BS * APAD)
    hitconf, = _sc_gather(conf_flat, targets_t)
    out = _reduce(targets_t, cand, hitconf, s1p)
    return out[0, 0]


# speedup vs baseline: 1.0057x; 1.0013x over previous
import jax
import jax.numpy as jnp
from jax import lax
from jax.experimental import pallas as pl
from jax.experimental.pallas import tpu as pltpu
from jax.experimental.pallas import tpu_sc as plsc

IMG = 608.0
NCLS = 80
NFM = 3
A = 3
FM = (19.0, 38.0, 76.0)
LAST = (0, 1083, 5415)
SCALED_ANCH = (
    ((3.625, 2.8125), (4.875, 6.1875), (11.65625, 10.1875)),
    ((1.875, 3.8125), (3.875, 2.8125), (3.6875, 7.4375)),
    ((1.25, 1.625), (2.0, 2.875), (4.125, 2.875)),
)
NT = 50
NPAD = 64
BS = 16
NANCH = 22743
NALL = BS * NANCH
APAD = 23552
BA = 1024
NHIT = A * NFM * NPAD


def _assign_vec(t1, t2, t3, t4, use_floor):
    best = None
    best_pos = None
    bases = []
    ious = []
    for m in range(NFM):
        fm = FM[m]
        gw = t3 * fm
        gh = t4 * fm
        if use_floor:
            gi = jnp.floor(t1 * fm).astype(jnp.int32)
            gj = jnp.floor(t2 * fm).astype(jnp.int32)
        else:
            gi = (t1 * fm).astype(jnp.int32)
            gj = (t2 * fm).astype(jnp.int32)
        base = LAST[m] + A * (gi * gj)
        bases.append(base)
        row = []
        for i in range(A):
            aw, ah = SCALED_ANCH[m][i]
            inter = jnp.minimum(gw, aw) * jnp.minimum(gh, ah)
            union = gw * gh + (aw * ah) - inter
            iou = inter / (union + 1e-16)
            row.append(iou)
            cand_pos = base + i
            if best is None:
                best, best_pos = iou, cand_pos
            else:
                upd = iou > best
                best = jnp.where(upd, iou, best)
                best_pos = jnp.where(upd, cand_pos, best_pos)
        ious.append(row)
    return best_pos, bases, best, ious


def _extract_kernel(targets_ref, pred_ref, conf_ref, cand_ref, s1p_ref):
    g = pl.program_id(0)
    f32 = jnp.float32
    data = pred_ref[...]
    a_iota = lax.broadcasted_iota(jnp.int32, (BS, BA), 1) + g * BA
    valid_a = a_iota < NANCH
    lane3 = lax.broadcasted_iota(jnp.int32, data.shape, 2)
    conf = jnp.sum(jnp.where(lane3 == 4, data, 0.0), axis=2)
    conf_m = jnp.where(valid_a, conf, 0.0)
    conf_ref[...] = conf_m.reshape(BS, BA // 128, 128)

    clipv = lambda p: jnp.clip(p, 1e-7, 1.0 - 1e-7)
    terms = jnp.where(valid_a, -jnp.log(1.0 - clipv(conf)), 0.0)
    t3 = terms.reshape(BS, BA // 128, 128)
    psum = jnp.sum(t3, axis=(0, 1))
    p8 = jnp.broadcast_to(psum[None, :], (8, 128)) * f32(0.125)

    targets_p = [targets_ref[c] for c in range(5)]
    pos, _, _, _ = _assign_vec(targets_p[1], targets_p[2], targets_p[3],
                               targets_p[4], True)

    a3 = lax.broadcasted_iota(jnp.int32, data.shape, 1) + g * BA
    dmask = jnp.where(a3 < NANCH, data, 0.0)

    @pl.when(g == 0)
    def _init():
        cand_ref[...] = jnp.zeros((BS, NPAD, 85), f32)
        s1p_ref[...] = jnp.zeros((8, 128), f32)

    s1p_ref[...] += p8
    a_row3 = lax.broadcasted_iota(jnp.int32, (BS, NPAD, BA), 2) + g * BA
    onehot = (a_row3 == pos[:, :, None]).astype(f32)
    cand_ref[...] += lax.dot_general(
        onehot, dmask,
        (((2,), (1,)), ((0,), (0,))),
        preferred_element_type=f32)


def _extract(targets_t, prediction):
    grid = -(-NANCH // BA)
    return pl.pallas_call(
        _extract_kernel,
        grid=(grid,),
        in_specs=[
            pl.BlockSpec((5, BS, NPAD), lambda g: (0, 0, 0)),
            pl.BlockSpec((BS, BA, 85), lambda g: (0, g, 0)),
        ],
        out_specs=[
            pl.BlockSpec((BS, BA // 128, 128), lambda g: (0, g, 0)),
            pl.BlockSpec((BS, NPAD, 85), lambda g: (0, 0, 0)),
            pl.BlockSpec((8, 128), lambda g: (0, 0)),
        ],
        out_shape=[
            jax.ShapeDtypeStruct((BS, APAD // 128, 128), jnp.float32),
            jax.ShapeDtypeStruct((BS, NPAD, 85), jnp.float32),
            jax.ShapeDtypeStruct((8, 128), jnp.float32),
        ],
    )(targets_t, prediction)


def _sc_gather_kernel(conf_hbm, targets_hbm, hitconf_out,
                      tgt_v, hitw_idx_v, hitw_v, sem):
    wid = lax.axis_index("s") * 2 + lax.axis_index("c")

    @pl.when(wid < BS)
    def _sparse():
        b = wid
        pltpu.sync_copy(targets_hbm.at[:, b, :], tgt_v)
        word0 = b * APAD
        pad = jnp.full((16,), 0, jnp.int32)
        for kpad in range(NHIT, 640, 16):
            hitw_idx_v[pl.ds(kpad, 16)] = pad
        for chunk in range(NPAD // 16):
            sl = pl.ds(chunk * 16, 16)
            t1 = tgt_v[1, sl]
            t2 = tgt_v[2, sl]
            t3 = tgt_v[3, sl]
            t4 = tgt_v[4, sl]
            _, bases, _, _ = _assign_vec(t1, t2, t3, t4, False)
            for m in range(NFM):
                for i in range(A):
                    k = (m * A + i) * NPAD + chunk * 16
                    hitw_idx_v[pl.ds(k, 16)] = (
                        word0 + jnp.clip(bases[m] + i, 0, NANCH - 1))
        pltpu.async_copy(conf_hbm.at[hitw_idx_v], hitw_v, sem).wait()
        pltpu.sync_copy(hitw_v, hitconf_out.at[b])


def _sc_gather(conf_flat, targets_t):
    mesh = plsc.VectorSubcoreMesh(core_axis_name="c", subcore_axis_name="s")
    kfn = pl.kernel(
        _sc_gather_kernel,
        mesh=mesh,
        compiler_params=pltpu.CompilerParams(use_tc_tiling_on_sc=False),
        out_type=[
            jax.ShapeDtypeStruct((BS, 640), jnp.float32),
        ],
        scratch_types=[
            pltpu.VMEM((5, NPAD), jnp.float32),
            pltpu.VMEM((640,), jnp.int32),
            pltpu.VMEM((640,), jnp.float32),
            pltpu.SemaphoreType.DMA,
        ],
    )
    return kfn(conf_flat, targets_t)


def _reduce_kernel(targets_ref, cand_ref, hitconf_ref, s1p_ref, out_ref):
    f32 = jnp.float32
    targets_p = [targets_ref[c] for c in range(5)]
    valid = (targets_p[0] + targets_p[1] + targets_p[2] +
             targets_p[3] + targets_p[4]) != 0.0
    pos, bases, tconf, ious = _assign_vec(
        targets_p[1], targets_p[2], targets_p[3], targets_p[4], True)
    cls = targets_p[0].astype(jnp.int32)

    clipv = lambda p: jnp.clip(p, 1e-7, 1.0 - 1e-7)
    C0 = -jnp.log(f32(1.0) - clipv(f32(0.0)))
    Nf = f32(NALL)

    n_iota2 = lax.broadcasted_iota(jnp.int32, (BS, NPAD, NPAD), 2)
    n_iota1 = lax.broadcasted_iota(jnp.int32, (BS, NPAD, NPAD), 1)
    eqpos = pos[:, :, None] == pos[:, None, :]
    overw = eqpos & (n_iota2 > n_iota1) & valid[:, None, :]
    is_last = valid & jnp.logical_not(jnp.any(overw, axis=2))
    eqcls = cls[:, :, None] == cls[:, None, :]
    earlier_same = eqpos & eqcls & (n_iota2 < n_iota1) & valid[:, None, :]
    cls_first = valid & jnp.logical_not(jnp.any(earlier_same, axis=2))
    wl = is_last.astype(f32)
    wcf = cls_first.astype(f32)

    cand = cand_ref[...]
    lane = lax.broadcasted_iota(jnp.int32, cand.shape, 2)
    pick = lambda c: jnp.sum(jnp.where(lane == c, cand, 0.0), axis=2)
    xg, yg, wg, hg, cg = pick(0), pick(1), pick(2), pick(3), pick(4)

    wwh = 2.0 - targets_p[3] * targets_p[4]
    tx = targets_p[1] * IMG
    ty = targets_p[2] * IMG
    tw = targets_p[3] * IMG
    th = targets_p[4] * IMG
    sxywh = jnp.sum(wl * ((xg * wwh - tx * wwh) ** 2 +
                          (yg * wwh - ty * wwh) ** 2 +
                          (wg * wwh - tw * wwh) ** 2 +
                          (hg * wwh - th * wwh) ** 2))

    cgc = clipv(cg)
    mask_corr = jnp.sum(wl * (-(tconf * jnp.log(cgc) +
                                (1.0 - tconf) * jnp.log(1.0 - cgc)) - C0))

    pc = clipv(cand)
    l1m = jnp.log(1.0 - pc)
    rowsum = jnp.sum(jnp.where(lane >= 5, -l1m, 0.0), axis=2)
    selmask = lane == (5 + cls)[:, :, None]
    selterm = jnp.sum(jnp.where(selmask, l1m - jnp.log(pc), 0.0), axis=2)
    clsnum = jnp.sum(wl * rowsum) + jnp.sum(wcf * selterm)
    nmask = jnp.sum(wl)

    S1 = jnp.sum(s1p_ref[...])

    noobj_corr = f32(0.0)
    for m in range(NFM):
        hitk = jnp.concatenate(
            [((ious[m][i] > 0.5) & valid).astype(jnp.int32)
             for i in range(A)], axis=1) > 0
        idxk = jnp.concatenate(
            [bases[m] + i for i in range(A)], axis=1)
        K = A * NPAD
        k2 = lax.broadcasted_iota(jnp.int32, (BS, K, K), 2)
        k1 = lax.broadcasted_iota(jnp.int32, (BS, K, K), 1)
        eqi = idxk[:, :, None] == idxk[:, None, :]
        earlier_hit = eqi & (k2 < k1) & hitk[:, None, :]
        hit_keep = hitk & jnp.logical_not(jnp.any(earlier_hit, axis=2))
        hconf = jnp.concatenate(
            [hitconf_ref[:, (m * A + i) * NPAD:(m * A + i + 1) * NPAD]
             for i in range(A)], axis=1)
        noobj_corr = noobj_corr + jnp.sum(
            hit_keep.astype(f32) * (C0 + jnp.log(1.0 - clipv(hconf))))

    loss = (sxywh / Nf
            + (Nf * C0 + mask_corr) / Nf
            + 0.5 * (S1 + noobj_corr) / Nf
            + clsnum / (nmask * NCLS))
    out_ref[...] = jnp.full((8, 128), loss, jnp.float32)


def _reduce(targets_t, cand, hitconf, s1p):
    return pl.pallas_call(
        _reduce_kernel,
        out_shape=jax.ShapeDtypeStruct((8, 128), jnp.float32),
    )(targets_t, cand, hitconf, s1p)


def kernel(prediction, targets, stride):
    targets_p = jnp.pad(targets, ((0, 0), (0, NPAD - NT), (0, 0)))
    targets_t = jnp.transpose(targets_p, (2, 0, 1))
    conf3, cand, s1p = _extract(targets_t, prediction)
    conf_flat = conf3.reshape(BS * APAD)
    hitconf, = _sc_gather(conf_flat, targets_t)
    out = _reduce(targets_t, cand, hitconf, s1p)
    return out[0, 0]
